# Initial kernel scaffold; baseline (speedup 1.0000x reference)
import jax
import jax.numpy as jnp
from jax.experimental import pallas as pl


def kernel(sparse_batch, dense_batch, stm_players, ft_W, ft_b, fc1_W, fc1_b, fc2v_W, fc2v_b, fc2p_W, fc2p_b):
    raise NotImplementedError("write your pallas kernel here")


# speedup vs baseline: 8.9995x; 8.9995x over previous
import functools

import jax
import jax.numpy as jnp
from jax import lax
from jax.experimental import pallas as pl

PIECE_HEX_DIM = 120
P1_FEATURE_CUTOFF = 60
DENSE_DIM = 66
HEX_COUNT = 60
FT_DIM = 256
HIDDEN_DIM = 512
B = 16384

LANE = 128
R = 1024


def _fwd(sp_ref, dense_ref, stm_ref, ftw_ref, ftb_ref,
         w1a_ref, w1b_ref, w1c_ref, b1_ref, w2_ref, b2_ref,
         pol_ref, val_ref):
    sp = sp_ref[...]
    rows = sp.shape[0]
    iota = lax.broadcasted_iota(jnp.int32, (rows, LANE), 1)
    zero = jnp.zeros((rows, LANE), jnp.int32)
    stm0 = (stm_ref[...] + zero) == 0

    c_stm = jnp.zeros((rows, LANE), jnp.float32)
    c_nstm = jnp.zeros((rows, LANE), jnp.float32)
    cum_s = jnp.zeros((rows, LANE), jnp.float32)
    cum_n = jnp.zeros((rows, LANE), jnp.float32)
    for j in range(6):
        spj = sp[:, j:j + 1] + zero
        eqf = jnp.where(spj == iota, 1.0, 0.0)
        p1 = spj < P1_FEATURE_CUTOFF
        m_s = jnp.where(jnp.logical_xor(stm0, p1), 0.0, 1.0)
        m_n = 1.0 - m_s
        c_stm = c_stm + jnp.where(cum_s < 3.0, m_s, 0.0) * eqf
        c_nstm = c_nstm + jnp.where(cum_n < 3.0, m_n, 0.0) * eqf
        cum_s = cum_s + m_s
        cum_n = cum_n + m_n
    c_stm = c_stm + jnp.where(iota == 0, 3.0 - jnp.minimum(cum_s, 3.0), 0.0)
    c_nstm = c_nstm + jnp.where(iota == 0, 3.0 - jnp.minimum(cum_n, 3.0), 0.0)

    ftw = ftw_ref[...]
    ftb = ftb_ref[...]
    acc_s = jnp.maximum(jnp.dot(c_stm, ftw, preferred_element_type=jnp.float32) + ftb, 0.0)
    acc_n = jnp.maximum(jnp.dot(c_nstm, ftw, preferred_element_type=jnp.float32) + ftb, 0.0)

    h = jnp.dot(acc_s, w1a_ref[...], preferred_element_type=jnp.float32)
    h = h + jnp.dot(acc_n, w1b_ref[...], preferred_element_type=jnp.float32)
    h = h + jnp.dot(dense_ref[...], w1c_ref[...], preferred_element_type=jnp.float32)
    h = jnp.maximum(h + b1_ref[...], 0.0)

    out = jnp.dot(h, w2_ref[...], preferred_element_type=jnp.float32) + b2_ref[...]
    pol_ref[...] = out[:, :HEX_COUNT]
    val_ref[...] = jnp.tanh(out[:, HEX_COUNT:HEX_COUNT + 1])


@jax.jit
def _run(sparse_batch, dense_pad, stm2d, ftw_pad, ftb2d,
         w1a, w1b, w1c, b12d, w2, b22d):
    grid = (B // R,)
    row = lambda i: (i, 0)
    rep = lambda i: (0, 0)
    pol, val = pl.pallas_call(
        _fwd,
        grid=grid,
        in_specs=[
            pl.BlockSpec((R, 6), row),
            pl.BlockSpec((R, LANE), row),
            pl.BlockSpec((R, 1), row),
            pl.BlockSpec((LANE, FT_DIM), rep),
            pl.BlockSpec((1, FT_DIM), rep),
            pl.BlockSpec((FT_DIM, HIDDEN_DIM), rep),
            pl.BlockSpec((FT_DIM, HIDDEN_DIM), rep),
            pl.BlockSpec((LANE, HIDDEN_DIM), rep),
            pl.BlockSpec((1, HIDDEN_DIM), rep),
            pl.BlockSpec((HIDDEN_DIM, 64), rep),
            pl.BlockSpec((1, 64), rep),
        ],
        out_specs=[
            pl.BlockSpec((R, HEX_COUNT), row),
            pl.BlockSpec((R, 1), row),
        ],
        out_shape=[
            jax.ShapeDtypeStruct((B, HEX_COUNT), jnp.float32),
            jax.ShapeDtypeStruct((B, 1), jnp.float32),
        ],
    )(sparse_batch, dense_pad, stm2d, ftw_pad, ftb2d,
      w1a, w1b, w1c, b12d, w2, b22d)
    return pol, val[:, 0]


def kernel(sparse_batch, dense_batch, stm_players, ft_W, ft_b,
           fc1_W, fc1_b, fc2v_W, fc2v_b, fc2p_W, fc2p_b):
    sp = sparse_batch.astype(jnp.int32)
    dense_pad = jnp.pad(dense_batch, ((0, 0), (0, LANE - DENSE_DIM)))
    stm2d = stm_players.astype(jnp.int32).reshape(B, 1)

    ftw_pad = jnp.pad(ft_W.T, ((0, LANE - PIECE_HEX_DIM), (0, 0)))
    ftb2d = ft_b.reshape(1, FT_DIM)

    w1t = fc1_W.T
    w1a = w1t[:FT_DIM]
    w1b = w1t[FT_DIM:2 * FT_DIM]
    w1c = jnp.pad(w1t[2 * FT_DIM:], ((0, LANE - DENSE_DIM), (0, 0)))
    b12d = fc1_b.reshape(1, HIDDEN_DIM)

    w2 = jnp.pad(jnp.concatenate([fc2p_W, fc2v_W], axis=0).T,
                 ((0, 0), (0, 64 - HEX_COUNT - 1)))
    b22d = jnp.pad(jnp.concatenate([fc2p_b, fc2v_b], axis=0),
                   (0, 64 - HEX_COUNT - 1)).reshape(1, 64)

    pol, val = _run(sp, dense_pad, stm2d, ftw_pad, ftb2d,
                    w1a, w1b, w1c, b12d, w2, b22d)
    return (pol, val)


# speedup vs baseline: 9.1728x; 1.0193x over previous
import functools

import jax
import jax.numpy as jnp
from jax import lax
from jax.experimental import pallas as pl

PIECE_HEX_DIM = 120
P1_FEATURE_CUTOFF = 60
DENSE_DIM = 66
HEX_COUNT = 60
FT_DIM = 256
HIDDEN_DIM = 512
B = 16384

LANE = 128
R = 1024


def _fwd(sp_ref, dense_ref, stm_ref, ftw_ref, ftb_ref,
         w1a_ref, w1b_ref, w1c_ref, b1_ref, w2_ref, b2_ref,
         pol_ref, val_ref):
    sp = sp_ref[...]
    rows = sp.shape[0]
    iota = lax.broadcasted_iota(jnp.int32, (rows, LANE), 1)
    zero = jnp.zeros((rows, LANE), jnp.int32)
    stm0 = (stm_ref[...] + zero) == 0

    c_stm = jnp.zeros((rows, LANE), jnp.float32)
    c_nstm = jnp.zeros((rows, LANE), jnp.float32)
    cum_s = jnp.zeros((rows, LANE), jnp.float32)
    cum_n = jnp.zeros((rows, LANE), jnp.float32)
    for j in range(6):
        spj = sp[:, j:j + 1] + zero
        eqf = jnp.where(spj == iota, 1.0, 0.0)
        p1 = spj < P1_FEATURE_CUTOFF
        m_s = jnp.where(jnp.logical_xor(stm0, p1), 0.0, 1.0)
        m_n = 1.0 - m_s
        c_stm = c_stm + jnp.where(cum_s < 3.0, m_s, 0.0) * eqf
        c_nstm = c_nstm + jnp.where(cum_n < 3.0, m_n, 0.0) * eqf
        cum_s = cum_s + m_s
        cum_n = cum_n + m_n
    c_stm = c_stm + jnp.where(iota == 0, 3.0 - jnp.minimum(cum_s, 3.0), 0.0)
    c_nstm = c_nstm + jnp.where(iota == 0, 3.0 - jnp.minimum(cum_n, 3.0), 0.0)

    ftw = ftw_ref[...]
    ftb = ftb_ref[...]
    c_stm = c_stm.astype(jnp.bfloat16)
    c_nstm = c_nstm.astype(jnp.bfloat16)
    acc_s = jnp.maximum(jnp.dot(c_stm, ftw, preferred_element_type=jnp.float32) + ftb, 0.0)
    acc_n = jnp.maximum(jnp.dot(c_nstm, ftw, preferred_element_type=jnp.float32) + ftb, 0.0)

    h = jnp.dot(acc_s.astype(jnp.bfloat16), w1a_ref[...], preferred_element_type=jnp.float32)
    h = h + jnp.dot(acc_n.astype(jnp.bfloat16), w1b_ref[...], preferred_element_type=jnp.float32)
    h = h + jnp.dot(dense_ref[...], w1c_ref[...], preferred_element_type=jnp.float32)
    h = jnp.maximum(h + b1_ref[...], 0.0)

    out = jnp.dot(h, w2_ref[...], preferred_element_type=jnp.float32) + b2_ref[...]
    pol_ref[...] = out[:, :HEX_COUNT]
    val_ref[...] = jnp.tanh(out[:, HEX_COUNT:HEX_COUNT + 1])


@jax.jit
def _run(sparse_batch, dense_pad, stm2d, ftw_pad, ftb2d,
         w1a, w1b, w1c, b12d, w2, b22d):
    grid = (B // R,)
    row = lambda i: (i, 0)
    rep = lambda i: (0, 0)
    pol, val = pl.pallas_call(
        _fwd,
        grid=grid,
        in_specs=[
            pl.BlockSpec((R, 6), row),
            pl.BlockSpec((R, LANE), row),
            pl.BlockSpec((R, 1), row),
            pl.BlockSpec((LANE, FT_DIM), rep),
            pl.BlockSpec((1, FT_DIM), rep),
            pl.BlockSpec((FT_DIM, HIDDEN_DIM), rep),
            pl.BlockSpec((FT_DIM, HIDDEN_DIM), rep),
            pl.BlockSpec((LANE, HIDDEN_DIM), rep),
            pl.BlockSpec((1, HIDDEN_DIM), rep),
            pl.BlockSpec((HIDDEN_DIM, 64), rep),
            pl.BlockSpec((1, 64), rep),
        ],
        out_specs=[
            pl.BlockSpec((R, HEX_COUNT), row),
            pl.BlockSpec((R, 1), row),
        ],
        out_shape=[
            jax.ShapeDtypeStruct((B, HEX_COUNT), jnp.float32),
            jax.ShapeDtypeStruct((B, 1), jnp.float32),
        ],
    )(sparse_batch, dense_pad, stm2d, ftw_pad, ftb2d,
      w1a, w1b, w1c, b12d, w2, b22d)
    return pol, val[:, 0]


def kernel(sparse_batch, dense_batch, stm_players, ft_W, ft_b,
           fc1_W, fc1_b, fc2v_W, fc2v_b, fc2p_W, fc2p_b):
    sp = sparse_batch.astype(jnp.int32)
    dense_pad = jnp.pad(dense_batch, ((0, 0), (0, LANE - DENSE_DIM))).astype(jnp.bfloat16)
    stm2d = stm_players.astype(jnp.int32).reshape(B, 1)

    ftw_pad = jnp.pad(ft_W.T, ((0, LANE - PIECE_HEX_DIM), (0, 0))).astype(jnp.bfloat16)
    ftb2d = ft_b.reshape(1, FT_DIM)

    w1t = fc1_W.T
    w1a = w1t[:FT_DIM].astype(jnp.bfloat16)
    w1b = w1t[FT_DIM:2 * FT_DIM].astype(jnp.bfloat16)
    w1c = jnp.pad(w1t[2 * FT_DIM:], ((0, LANE - DENSE_DIM), (0, 0))).astype(jnp.bfloat16)
    b12d = fc1_b.reshape(1, HIDDEN_DIM)

    w2 = jnp.pad(jnp.concatenate([fc2p_W, fc2v_W], axis=0).T,
                 ((0, 0), (0, 64 - HEX_COUNT - 1)))
    b22d = jnp.pad(jnp.concatenate([fc2p_b, fc2v_b], axis=0),
                   (0, 64 - HEX_COUNT - 1)).reshape(1, 64)

    pol, val = _run(sp, dense_pad, stm2d, ftw_pad, ftb2d,
                    w1a, w1b, w1c, b12d, w2, b22d)
    return (pol, val)


# speedup vs baseline: 16.8568x; 1.8377x over previous
import jax
import jax.numpy as jnp
from jax import lax
from jax.experimental import pallas as pl

PIECE_HEX_DIM = 120
P1_FEATURE_CUTOFF = 60
DENSE_DIM = 66
HEX_COUNT = 60
FT_DIM = 256
HIDDEN_DIM = 512
B = 16384

LANE = 128
DPAD = 72
R = 1024
OUT_ROWS = 64


def _fwd(sp_ref, dense_ref, ftw_ref, ftb_ref,
         w1a_ref, w1b_ref, w1c_ref, b1_ref, w2_ref, b2_ref,
         out_ref):
    cols = sp_ref.shape[1]
    stm0 = sp_ref[6:7, :] == 0.0
    iota_bf = lax.broadcasted_iota(
        jnp.int32, (LANE, cols), 0).astype(jnp.bfloat16)
    pad_row = iota_bf == 0.0

    counts = [jnp.zeros((LANE, cols), jnp.bfloat16) for _ in range(2)]
    cum = [jnp.zeros((1, cols), jnp.float32) for _ in range(2)]
    one = jnp.ones((LANE, cols), jnp.bfloat16)
    zero = jnp.zeros((LANE, cols), jnp.bfloat16)
    for j in range(6):
        spj = sp_ref[j:j + 1, :]
        p1 = spj < P1_FEATURE_CUTOFF
        m = [stm0 == p1, stm0 != p1]
        for t in range(2):
            sel = m[t] & (cum[t] < 3.0)
            spx = jnp.where(sel, spj, -1.0).astype(jnp.bfloat16)
            counts[t] = counts[t] + jnp.where(spx == iota_bf, one, zero)
            cum[t] = cum[t] + jnp.where(m[t], 1.0, 0.0)
    for t in range(2):
        padn = (3.0 - jnp.minimum(cum[t], 3.0)).astype(jnp.bfloat16)
        counts[t] = counts[t] + jnp.where(pad_row, padn, zero)

    ftw = ftw_ref[...]
    ftb = ftb_ref[...]
    acc_s = jnp.maximum(
        jnp.dot(ftw, counts[0], preferred_element_type=jnp.float32) + ftb, 0.0)
    acc_n = jnp.maximum(
        jnp.dot(ftw, counts[1], preferred_element_type=jnp.float32) + ftb, 0.0)

    h = jnp.dot(w1a_ref[...], acc_s.astype(jnp.bfloat16),
                preferred_element_type=jnp.float32)
    h = h + jnp.dot(w1b_ref[...], acc_n.astype(jnp.bfloat16),
                    preferred_element_type=jnp.float32)
    h = h + jnp.dot(w1c_ref[...], dense_ref[...],
                    preferred_element_type=jnp.float32)
    h = jnp.maximum(h + b1_ref[...], 0.0)

    out = jnp.dot(w2_ref[...], h, preferred_element_type=jnp.float32)
    out = out + b2_ref[...]
    vrow = lax.broadcasted_iota(jnp.int32, (OUT_ROWS, cols), 0) == HEX_COUNT
    out_ref[...] = jnp.where(vrow, jnp.tanh(out), out)


@jax.jit
def _run(sp_t, dense_t, ftw, ftb, w1a, w1b, w1c, b1, w2, b2):
    grid = (B // R,)
    col = lambda i: (0, i)
    rep = lambda i: (0, 0)
    out = pl.pallas_call(
        _fwd,
        grid=grid,
        in_specs=[
            pl.BlockSpec((8, R), col),
            pl.BlockSpec((DPAD, R), col),
            pl.BlockSpec((FT_DIM, LANE), rep),
            pl.BlockSpec((FT_DIM, 1), rep),
            pl.BlockSpec((HIDDEN_DIM, FT_DIM), rep),
            pl.BlockSpec((HIDDEN_DIM, FT_DIM), rep),
            pl.BlockSpec((HIDDEN_DIM, DPAD), rep),
            pl.BlockSpec((HIDDEN_DIM, 1), rep),
            pl.BlockSpec((OUT_ROWS, HIDDEN_DIM), rep),
            pl.BlockSpec((OUT_ROWS, 1), rep),
        ],
        out_specs=pl.BlockSpec((OUT_ROWS, R), col),
        out_shape=jax.ShapeDtypeStruct((OUT_ROWS, B), jnp.float32),
    )(sp_t, dense_t, ftw, ftb, w1a, w1b, w1c, b1, w2, b2)
    return out


def kernel(sparse_batch, dense_batch, stm_players, ft_W, ft_b,
           fc1_W, fc1_b, fc2v_W, fc2v_b, fc2p_W, fc2p_b):
    sp_t = jnp.concatenate(
        [sparse_batch.astype(jnp.float32).T,
         stm_players.astype(jnp.float32).reshape(1, B),
         jnp.zeros((1, B), jnp.float32)], axis=0)
    dense_t = jnp.pad(dense_batch.T.astype(jnp.bfloat16),
                      ((0, DPAD - DENSE_DIM), (0, 0)))

    ftw = jnp.pad(ft_W, ((0, 0), (0, LANE - PIECE_HEX_DIM))
                  ).astype(jnp.bfloat16)
    ftb = ft_b.reshape(FT_DIM, 1)

    w1a = fc1_W[:, :FT_DIM].astype(jnp.bfloat16)
    w1b = fc1_W[:, FT_DIM:2 * FT_DIM].astype(jnp.bfloat16)
    w1c = jnp.pad(fc1_W[:, 2 * FT_DIM:],
                  ((0, 0), (0, DPAD - DENSE_DIM))).astype(jnp.bfloat16)
    b1 = fc1_b.reshape(HIDDEN_DIM, 1)

    w2 = jnp.pad(jnp.concatenate([fc2p_W, fc2v_W], axis=0),
                 ((0, OUT_ROWS - HEX_COUNT - 1), (0, 0)))
    b2 = jnp.pad(jnp.concatenate([fc2p_b, fc2v_b], axis=0),
                 (0, OUT_ROWS - HEX_COUNT - 1)).reshape(OUT_ROWS, 1)

    out = _run(sp_t, dense_t, ftw, ftb, w1a, w1b, w1c, b1, w2, b2)
    return (out[:HEX_COUNT].T, out[HEX_COUNT])
